# Initial kernel scaffold; baseline (speedup 1.0000x reference)
import jax
import jax.numpy as jnp
from jax.experimental import pallas as pl


def kernel(hidden_states, ln1_w, ln2_w, Wq, bq, Wk, bk, Wv, bv, Wo, W_route, W_noise, W_gate, W_up, W_down, lora_A, lora_B):
    raise NotImplementedError("write your pallas kernel here")


# speedup vs baseline: 1.0024x; 1.0024x over previous
import functools

import jax
import jax.numpy as jnp
import numpy as np
from jax.experimental import pallas as pl
from jax.experimental.pallas import tpu as pltpu

S = 2048
HIDDEN = 1024
HEADS = 16
HEAD_DIM = 64
FFN = 2816
NUM_EXPERTS = 8
TOP_K = 2
LORA_R = 16
LORA_SCALING = 2.0
RMS_EPS = 1e-6
ROPE_THETA = 10000.0

BLK = 256
Q_BLK = 256
KV_BLK = 256

NEG_INF = -1e30


def _rms(x32, w):
    var = jnp.mean(x32 * x32, axis=-1, keepdims=True)
    return (x32 * jax.lax.rsqrt(var + RMS_EPS)) * w


def _qkv_body(h_ref, ln1_ref, wq_ref, wk_ref, wv_ref, bq_ref, bk_ref, bv_ref,
              cos_ref, sin_ref, q_out, k_out, v_out):
    h = h_ref[...]
    x = _rms(h, ln1_ref[...]).astype(jnp.bfloat16)

    def proj(w_ref, b_ref):
        y = jax.lax.dot_general(x, w_ref[...], (((1,), (0,)), ((), ())),
                                preferred_element_type=jnp.float32)
        y = y + b_ref[...]
        return y.reshape(BLK, HEADS, HEAD_DIM).transpose(1, 0, 2)

    cos = cos_ref[...][None, :, :]
    sin = sin_ref[...][None, :, :]

    def rope(t):
        t1 = t[..., : HEAD_DIM // 2]
        t2 = t[..., HEAD_DIM // 2:]
        rot = jnp.concatenate([-t2, t1], axis=-1)
        return t * cos + rot * sin

    q = proj(wq_ref, bq_ref)
    k = proj(wk_ref, bk_ref)
    v = proj(wv_ref, bv_ref)
    q_out[...] = rope(q).astype(jnp.bfloat16)
    k_out[...] = rope(k).astype(jnp.bfloat16)
    v_out[...] = v.astype(jnp.bfloat16)


def _flash_body(q_ref, k_ref, v_ref, o_ref):
    i = pl.program_id(1)
    q = q_ref[0]
    scale = 1.0 / np.sqrt(HEAD_DIM)

    row_ids = i * Q_BLK + jax.lax.broadcasted_iota(
        jnp.int32, (Q_BLK, KV_BLK), 0)

    def body(j, carry):
        acc, m, l = carry
        kb = k_ref[0, pl.ds(j * KV_BLK, KV_BLK), :]
        vb = v_ref[0, pl.ds(j * KV_BLK, KV_BLK), :]
        s = jax.lax.dot_general(q, kb, (((1,), (1,)), ((), ())),
                                preferred_element_type=jnp.float32) * scale
        col_ids = j * KV_BLK + jax.lax.broadcasted_iota(
            jnp.int32, (Q_BLK, KV_BLK), 1)
        s = jnp.where(row_ids >= col_ids, s, NEG_INF)
        m_new = jnp.maximum(m, jnp.max(s, axis=1, keepdims=True))
        alpha = jnp.exp(m - m_new)
        p = jnp.exp(s - m_new)
        l = l * alpha + jnp.sum(p, axis=1, keepdims=True)
        pv = jax.lax.dot_general(p.astype(jnp.bfloat16), vb,
                                 (((1,), (0,)), ((), ())),
                                 preferred_element_type=jnp.float32)
        acc = acc * alpha + pv
        return acc, m_new, l

    acc = jnp.zeros((Q_BLK, HEAD_DIM), jnp.float32)
    m0 = jnp.full((Q_BLK, 1), NEG_INF, jnp.float32)
    l0 = jnp.zeros((Q_BLK, 1), jnp.float32)
    acc, m, l = jax.lax.fori_loop(0, i + 1, body, (acc, m0, l0))
    o_ref[0] = (acc / l).astype(jnp.bfloat16)


def _moe_body(attn_ref, hid_ref, ln2_ref, wo_ref, wroute_ref, wnoise_ref,
              eps_ref, wg_ref, wu_ref, wd_ref, a2_ref, b2_ref,
              out_ref, rl_ref):
    attn = attn_ref[...].transpose(1, 0, 2).reshape(BLK, HIDDEN)
    ao = jax.lax.dot_general(attn, wo_ref[...], (((1,), (0,)), ((), ())),
                             preferred_element_type=jnp.float32)
    h = hid_ref[...] + ao

    x32 = _rms(h, ln2_ref[...])
    xb = x32.astype(jnp.bfloat16)

    logits = jax.lax.dot_general(x32, wroute_ref[...], (((1,), (0,)), ((), ())),
                                 preferred_element_type=jnp.float32)
    nz = jax.lax.dot_general(x32, wnoise_ref[...], (((1,), (0,)), ((), ())),
                             preferred_element_type=jnp.float32)
    rl = logits + eps_ref[...] * jax.nn.softplus(nz)
    rl_ref[...] = rl

    iota_e = jax.lax.broadcasted_iota(jnp.int32, (BLK, NUM_EXPERTS), 1)
    m1 = jnp.max(rl, axis=1, keepdims=True)
    i1 = jnp.min(jnp.where(rl == m1, iota_e, NUM_EXPERTS), axis=1,
                 keepdims=True)
    mask1 = iota_e == i1
    rl2 = jnp.where(mask1, NEG_INF, rl)
    m2 = jnp.max(rl2, axis=1, keepdims=True)
    i2 = jnp.min(jnp.where(rl2 == m2, iota_e, NUM_EXPERTS), axis=1,
                 keepdims=True)
    mask2 = iota_e == i2
    w1 = jax.nn.sigmoid(m1 - m2)
    w_dense = jnp.where(mask1, w1, 0.0) + jnp.where(mask2, 1.0 - w1, 0.0)

    g = jax.lax.dot_general(xb, wg_ref[...], (((1,), (0,)), ((), ())),
                            preferred_element_type=jnp.float32)
    u = jax.lax.dot_general(xb, wu_ref[...], (((1,), (0,)), ((), ())),
                            preferred_element_type=jnp.float32)
    s = (g * jax.nn.sigmoid(g) * u).astype(jnp.bfloat16)
    shared = jax.lax.dot_general(s, wd_ref[...], (((1,), (0,)), ((), ())),
                                 preferred_element_type=jnp.float32)

    mid = jax.lax.dot_general(xb, a2_ref[...], (((1,), (0,)), ((), ())),
                              preferred_element_type=jnp.float32)
    lane_e = jax.lax.broadcasted_iota(
        jnp.int32, (NUM_EXPERTS, NUM_EXPERTS * LORA_R), 1) // LORA_R
    row_e = jax.lax.broadcasted_iota(
        jnp.int32, (NUM_EXPERTS, NUM_EXPERTS * LORA_R), 0)
    expand = (lane_e == row_e).astype(jnp.float32)
    w128 = jax.lax.dot_general(w_dense, expand, (((1,), (0,)), ((), ())),
                               preferred_element_type=jnp.float32)
    wmid = (mid * w128).astype(jnp.bfloat16)
    lora = jax.lax.dot_general(wmid, b2_ref[...], (((1,), (0,)), ((), ())),
                               preferred_element_type=jnp.float32)

    out_ref[...] = h + shared + LORA_SCALING * lora


def _full_spec(shape):
    return pl.BlockSpec(shape, lambda *_: tuple(0 for _ in shape))


@jax.jit
def kernel(hidden_states, ln1_w, ln2_w, Wq, bq, Wk, bk, Wv, bv, Wo,
           W_route, W_noise, W_gate, W_up, W_down, lora_A, lora_B):
    Bsz, Sq, D = hidden_states.shape
    h2d = hidden_states.reshape(Sq, D)
    bf = jnp.bfloat16

    inv_freq = 1.0 / (ROPE_THETA ** (
        jnp.arange(0, HEAD_DIM, 2, dtype=jnp.float32) / HEAD_DIM))
    t = jnp.arange(Sq, dtype=jnp.float32)
    freqs = jnp.outer(t, inv_freq)
    emb = jnp.concatenate([freqs, freqs], axis=-1)
    cos, sin = jnp.cos(emb), jnp.sin(emb)
    eps = jax.random.normal(jax.random.key(1234), (Sq, NUM_EXPERTS),
                            dtype=jnp.float32)

    a2 = lora_A.transpose(1, 0, 2).reshape(HIDDEN, NUM_EXPERTS * LORA_R)
    b2 = lora_B.reshape(NUM_EXPERTS * LORA_R, HIDDEN)

    nblk = Sq // BLK
    q, k, v = pl.pallas_call(
        _qkv_body,
        grid=(nblk,),
        in_specs=[
            pl.BlockSpec((BLK, HIDDEN), lambda i: (i, 0)),
            _full_spec((HIDDEN,)),
            _full_spec((HIDDEN, HEADS * HEAD_DIM)),
            _full_spec((HIDDEN, HEADS * HEAD_DIM)),
            _full_spec((HIDDEN, HEADS * HEAD_DIM)),
            _full_spec((HEADS * HEAD_DIM,)),
            _full_spec((HEADS * HEAD_DIM,)),
            _full_spec((HEADS * HEAD_DIM,)),
            pl.BlockSpec((BLK, HEAD_DIM), lambda i: (i, 0)),
            pl.BlockSpec((BLK, HEAD_DIM), lambda i: (i, 0)),
        ],
        out_specs=[
            pl.BlockSpec((HEADS, BLK, HEAD_DIM), lambda i: (0, i, 0)),
            pl.BlockSpec((HEADS, BLK, HEAD_DIM), lambda i: (0, i, 0)),
            pl.BlockSpec((HEADS, BLK, HEAD_DIM), lambda i: (0, i, 0)),
        ],
        out_shape=[jax.ShapeDtypeStruct((HEADS, Sq, HEAD_DIM), bf)] * 3,
        compiler_params=pltpu.CompilerParams(
            dimension_semantics=("arbitrary",)),
    )(h2d, ln1_w, Wq.astype(bf), Wk.astype(bf), Wv.astype(bf),
      bq, bk, bv, cos, sin)

    attn = pl.pallas_call(
        _flash_body,
        grid=(HEADS, Sq // Q_BLK),
        in_specs=[
            pl.BlockSpec((1, Q_BLK, HEAD_DIM), lambda h, i: (h, i, 0)),
            pl.BlockSpec((1, Sq, HEAD_DIM), lambda h, i: (h, 0, 0)),
            pl.BlockSpec((1, Sq, HEAD_DIM), lambda h, i: (h, 0, 0)),
        ],
        out_specs=pl.BlockSpec((1, Q_BLK, HEAD_DIM), lambda h, i: (h, i, 0)),
        out_shape=jax.ShapeDtypeStruct((HEADS, Sq, HEAD_DIM), bf),
        compiler_params=pltpu.CompilerParams(
            dimension_semantics=("parallel", "arbitrary")),
    )(q, k, v)

    out2d, router_logits = pl.pallas_call(
        _moe_body,
        grid=(nblk,),
        in_specs=[
            pl.BlockSpec((HEADS, BLK, HEAD_DIM), lambda i: (0, i, 0)),
            pl.BlockSpec((BLK, HIDDEN), lambda i: (i, 0)),
            _full_spec((HIDDEN,)),
            _full_spec((HEADS * HEAD_DIM, HIDDEN)),
            _full_spec((HIDDEN, NUM_EXPERTS)),
            _full_spec((HIDDEN, NUM_EXPERTS)),
            pl.BlockSpec((BLK, NUM_EXPERTS), lambda i: (i, 0)),
            _full_spec((HIDDEN, FFN)),
            _full_spec((HIDDEN, FFN)),
            _full_spec((FFN, HIDDEN)),
            _full_spec((HIDDEN, NUM_EXPERTS * LORA_R)),
            _full_spec((NUM_EXPERTS * LORA_R, HIDDEN)),
        ],
        out_specs=[
            pl.BlockSpec((BLK, HIDDEN), lambda i: (i, 0)),
            pl.BlockSpec((BLK, NUM_EXPERTS), lambda i: (i, 0)),
        ],
        out_shape=[
            jax.ShapeDtypeStruct((Sq, HIDDEN), jnp.float32),
            jax.ShapeDtypeStruct((Sq, NUM_EXPERTS), jnp.float32),
        ],
        compiler_params=pltpu.CompilerParams(
            dimension_semantics=("arbitrary",)),
    )(attn, h2d, ln2_w, Wo.astype(bf), W_route, W_noise, eps,
      W_gate.astype(bf), W_up.astype(bf), W_down.astype(bf),
      a2.astype(bf), b2.astype(bf))

    return out2d.reshape(Bsz, Sq, D), router_logits


# speedup vs baseline: 1.0126x; 1.0102x over previous
import functools

import jax
import jax.numpy as jnp
import numpy as np
from jax.experimental import pallas as pl
from jax.experimental.pallas import tpu as pltpu

S = 2048
HIDDEN = 1024
HEADS = 16
HEAD_DIM = 64
FFN = 2816
NUM_EXPERTS = 8
TOP_K = 2
LORA_R = 16
LORA_SCALING = 2.0
RMS_EPS = 1e-6
ROPE_THETA = 10000.0

BLK = 256
Q_BLK = 256
KV_BLK = 256

NEG_INF = -1e30
LOG2E = 1.4426950408889634


def _rms(x32, w):
    var = jnp.mean(x32 * x32, axis=-1, keepdims=True)
    return (x32 * jax.lax.rsqrt(var + RMS_EPS)) * w


def _mm(a, b):
    return jax.lax.dot_general(a, b, (((1,), (0,)), ((), ())),
                               preferred_element_type=jnp.float32)


def _qkv_body(h_ref, ln1_ref, wq_ref, wqr_ref, wk_ref, wkr_ref, wv_ref,
              bq_ref, bqr_ref, bk_ref, bkr_ref, bv_ref,
              cos_ref, sin_ref, q_out, k_out, v_out):
    h = h_ref[...]
    x = _rms(h, ln1_ref[...]).astype(jnp.bfloat16)
    cos = cos_ref[...]
    sin = sin_ref[...]

    qa = _mm(x, wq_ref[...]) + bq_ref[...]
    qb = _mm(x, wqr_ref[...]) + bqr_ref[...]
    q = (qa * cos + qb * sin).astype(jnp.bfloat16)

    ka = _mm(x, wk_ref[...]) + bk_ref[...]
    kb = _mm(x, wkr_ref[...]) + bkr_ref[...]
    k = (ka * cos + kb * sin).astype(jnp.bfloat16)

    v = (_mm(x, wv_ref[...]) + bv_ref[...]).astype(jnp.bfloat16)

    for hp in range(HEADS // 2):
        sl = slice(hp * 128, hp * 128 + 128)
        q_out[hp] = q[:, sl]
        k_out[hp] = k[:, sl]
        v_out[hp] = v[:, sl]


def _flash_body(q_ref, k_ref, v_ref, o_ref):
    i = pl.program_id(1)
    q2 = (q_ref[0].astype(jnp.float32)
          * (LOG2E / np.sqrt(HEAD_DIM))).astype(jnp.bfloat16)
    qa = q2[:, :HEAD_DIM]
    qb = q2[:, HEAD_DIM:]

    def step(j, carry, masked):
        acc_a, m_a, l_a, acc_b, m_b, l_b = carry
        kb2 = k_ref[0, pl.ds(j * KV_BLK, KV_BLK), :]
        vb2 = v_ref[0, pl.ds(j * KV_BLK, KV_BLK), :]
        if masked:
            rows = jax.lax.broadcasted_iota(jnp.int32, (Q_BLK, KV_BLK), 0)
            cols = jax.lax.broadcasted_iota(jnp.int32, (Q_BLK, KV_BLK), 1)
            keep = rows >= cols

        def half(qh, ksl, vsl, acc, m, l):
            s = jax.lax.dot_general(qh, kb2[:, ksl],
                                    (((1,), (1,)), ((), ())),
                                    preferred_element_type=jnp.float32)
            if masked:
                s = jnp.where(keep, s, NEG_INF)
            m_new = jnp.maximum(m, jnp.max(s, axis=1, keepdims=True))
            alpha = jnp.exp2(m - m_new)
            p = jnp.exp2(s - m_new)
            l = l * alpha + jnp.sum(p, axis=1, keepdims=True)
            pv = jax.lax.dot_general(p.astype(jnp.bfloat16), vb2[:, vsl],
                                     (((1,), (0,)), ((), ())),
                                     preferred_element_type=jnp.float32)
            return acc * alpha + pv, m_new, l

        lo = slice(0, HEAD_DIM)
        hi = slice(HEAD_DIM, 2 * HEAD_DIM)
        acc_a, m_a, l_a = half(qa, lo, lo, acc_a, m_a, l_a)
        acc_b, m_b, l_b = half(qb, hi, hi, acc_b, m_b, l_b)
        return acc_a, m_a, l_a, acc_b, m_b, l_b

    z = jnp.zeros((Q_BLK, HEAD_DIM), jnp.float32)
    mi = jnp.full((Q_BLK, 1), NEG_INF, jnp.float32)
    zl = jnp.zeros((Q_BLK, 1), jnp.float32)
    carry = jax.lax.fori_loop(0, i, functools.partial(step, masked=False),
                              (z, mi, zl, z, mi, zl))
    acc_a, m_a, l_a, acc_b, m_b, l_b = step(i, carry, masked=True)
    o_ref[0] = jnp.concatenate(
        [acc_a / l_a, acc_b / l_b], axis=1).astype(jnp.bfloat16)


def _moe_body(attn_ref, hid_ref, ln2_ref, wo_ref, wroute_ref, wnoise_ref,
              eps_ref, wg_ref, wu_ref, wd_ref, a2_ref, b2_ref,
              out_ref, rl_ref):
    ao = _mm(attn_ref[0], wo_ref[pl.ds(0, 128), :])
    for hp in range(1, HEADS // 2):
        ao = ao + _mm(attn_ref[hp], wo_ref[pl.ds(hp * 128, 128), :])
    h = hid_ref[...] + ao

    x32 = _rms(h, ln2_ref[...])
    xb = x32.astype(jnp.bfloat16)

    logits = _mm(x32, wroute_ref[...])
    nz = _mm(x32, wnoise_ref[...])
    rl = logits + eps_ref[...] * jax.nn.softplus(nz)
    rl_ref[...] = rl

    iota_e = jax.lax.broadcasted_iota(jnp.int32, (BLK, NUM_EXPERTS), 1)
    m1 = jnp.max(rl, axis=1, keepdims=True)
    i1 = jnp.min(jnp.where(rl == m1, iota_e, NUM_EXPERTS), axis=1,
                 keepdims=True)
    mask1 = iota_e == i1
    rl2 = jnp.where(mask1, NEG_INF, rl)
    m2 = jnp.max(rl2, axis=1, keepdims=True)
    i2 = jnp.min(jnp.where(rl2 == m2, iota_e, NUM_EXPERTS), axis=1,
                 keepdims=True)
    mask2 = iota_e == i2
    w1 = jax.nn.sigmoid(m1 - m2)
    w_dense = jnp.where(mask1, w1, 0.0) + jnp.where(mask2, 1.0 - w1, 0.0)

    g = _mm(xb, wg_ref[...])
    u = _mm(xb, wu_ref[...])
    s = (g * jax.nn.sigmoid(g) * u).astype(jnp.bfloat16)
    shared = _mm(s, wd_ref[...])

    mid = _mm(xb, a2_ref[...])
    lane_e = jax.lax.broadcasted_iota(
        jnp.int32, (NUM_EXPERTS, NUM_EXPERTS * LORA_R), 1) // LORA_R
    row_e = jax.lax.broadcasted_iota(
        jnp.int32, (NUM_EXPERTS, NUM_EXPERTS * LORA_R), 0)
    expand = (lane_e == row_e).astype(jnp.float32)
    w128 = _mm(w_dense, expand)
    wmid = (mid * w128).astype(jnp.bfloat16)
    lora = _mm(wmid, b2_ref[...])

    out_ref[...] = h + shared + LORA_SCALING * lora


def _full_spec(shape):
    return pl.BlockSpec(shape, lambda *_: tuple(0 for _ in shape))


def _rot_cols(w):
    w3 = w.reshape(-1, HEADS, HEAD_DIM)
    return jnp.concatenate(
        [-w3[..., HEAD_DIM // 2:], w3[..., : HEAD_DIM // 2]],
        axis=-1).reshape(w.shape)


@jax.jit
def kernel(hidden_states, ln1_w, ln2_w, Wq, bq, Wk, bk, Wv, bv, Wo,
           W_route, W_noise, W_gate, W_up, W_down, lora_A, lora_B):
    Bsz, Sq, D = hidden_states.shape
    h2d = hidden_states.reshape(Sq, D)
    bf = jnp.bfloat16

    inv_freq = 1.0 / (ROPE_THETA ** (
        jnp.arange(0, HEAD_DIM, 2, dtype=jnp.float32) / HEAD_DIM))
    t = jnp.arange(Sq, dtype=jnp.float32)
    freqs = jnp.outer(t, inv_freq)
    emb = jnp.concatenate([freqs, freqs], axis=-1)
    cos = jnp.tile(jnp.cos(emb), (1, HEADS))
    sin = jnp.tile(jnp.sin(emb), (1, HEADS))
    eps = jax.random.normal(jax.random.key(1234), (Sq, NUM_EXPERTS),
                            dtype=jnp.float32)

    a2 = lora_A.transpose(1, 0, 2).reshape(HIDDEN, NUM_EXPERTS * LORA_R)
    b2 = lora_B.reshape(NUM_EXPERTS * LORA_R, HIDDEN)

    nblk = Sq // BLK
    nd = HEADS * HEAD_DIM
    q, k, v = pl.pallas_call(
        _qkv_body,
        grid=(nblk,),
        in_specs=[
            pl.BlockSpec((BLK, HIDDEN), lambda i: (i, 0)),
            _full_spec((HIDDEN,)),
            _full_spec((HIDDEN, nd)),
            _full_spec((HIDDEN, nd)),
            _full_spec((HIDDEN, nd)),
            _full_spec((HIDDEN, nd)),
            _full_spec((HIDDEN, nd)),
            _full_spec((nd,)),
            _full_spec((nd,)),
            _full_spec((nd,)),
            _full_spec((nd,)),
            _full_spec((nd,)),
            pl.BlockSpec((BLK, nd), lambda i: (i, 0)),
            pl.BlockSpec((BLK, nd), lambda i: (i, 0)),
        ],
        out_specs=[
            pl.BlockSpec((HEADS // 2, BLK, 128), lambda i: (0, i, 0)),
            pl.BlockSpec((HEADS // 2, BLK, 128), lambda i: (0, i, 0)),
            pl.BlockSpec((HEADS // 2, BLK, 128), lambda i: (0, i, 0)),
        ],
        out_shape=[jax.ShapeDtypeStruct((HEADS // 2, Sq, 128), bf)] * 3,
        compiler_params=pltpu.CompilerParams(
            dimension_semantics=("arbitrary",)),
    )(h2d, ln1_w, Wq.astype(bf), _rot_cols(Wq).astype(bf),
      Wk.astype(bf), _rot_cols(Wk).astype(bf), Wv.astype(bf),
      bq, _rot_cols(bq.reshape(1, nd)).reshape(nd),
      bk, _rot_cols(bk.reshape(1, nd)).reshape(nd), bv, cos, sin)

    attn = pl.pallas_call(
        _flash_body,
        grid=(HEADS // 2, Sq // Q_BLK),
        in_specs=[
            pl.BlockSpec((1, Q_BLK, 128), lambda h, i: (h, i, 0)),
            pl.BlockSpec((1, Sq, 128), lambda h, i: (h, 0, 0)),
            pl.BlockSpec((1, Sq, 128), lambda h, i: (h, 0, 0)),
        ],
        out_specs=pl.BlockSpec((1, Q_BLK, 128), lambda h, i: (h, i, 0)),
        out_shape=jax.ShapeDtypeStruct((HEADS // 2, Sq, 128), bf),
        compiler_params=pltpu.CompilerParams(
            dimension_semantics=("parallel", "arbitrary")),
    )(q, k, v)

    out2d, router_logits = pl.pallas_call(
        _moe_body,
        grid=(nblk,),
        in_specs=[
            pl.BlockSpec((HEADS // 2, BLK, 128), lambda i: (0, i, 0)),
            pl.BlockSpec((BLK, HIDDEN), lambda i: (i, 0)),
            _full_spec((HIDDEN,)),
            _full_spec((nd, HIDDEN)),
            _full_spec((HIDDEN, NUM_EXPERTS)),
            _full_spec((HIDDEN, NUM_EXPERTS)),
            pl.BlockSpec((BLK, NUM_EXPERTS), lambda i: (i, 0)),
            _full_spec((HIDDEN, FFN)),
            _full_spec((HIDDEN, FFN)),
            _full_spec((FFN, HIDDEN)),
            _full_spec((HIDDEN, NUM_EXPERTS * LORA_R)),
            _full_spec((NUM_EXPERTS * LORA_R, HIDDEN)),
        ],
        out_specs=[
            pl.BlockSpec((BLK, HIDDEN), lambda i: (i, 0)),
            pl.BlockSpec((BLK, NUM_EXPERTS), lambda i: (i, 0)),
        ],
        out_shape=[
            jax.ShapeDtypeStruct((Sq, HIDDEN), jnp.float32),
            jax.ShapeDtypeStruct((Sq, NUM_EXPERTS), jnp.float32),
        ],
        compiler_params=pltpu.CompilerParams(
            dimension_semantics=("arbitrary",)),
    )(attn, h2d, ln2_w, Wo.astype(bf), W_route, W_noise, eps,
      W_gate.astype(bf), W_up.astype(bf), W_down.astype(bf),
      a2.astype(bf), b2.astype(bf))

    return out2d.reshape(Bsz, Sq, D), router_logits
